# baseline (speedup 1.0000x reference)
import functools

import jax
import jax.numpy as jnp
from jax import lax
from jax.experimental import pallas as pl
from jax.experimental.pallas import tpu as pltpu
from jax.experimental.pallas import tpu_sc as plsc

VOCAB_SIZE = 100000
EMBED_DIM = 64
NUM_CORES = 2
NUM_SUBCORES = 16
NUM_WORKERS = NUM_CORES * NUM_SUBCORES
TOTAL_ROWS = 4096 * 50
CHUNK = 128
ROWS_PER_WORKER = TOTAL_ROWS // NUM_WORKERS
CHUNKS_PER_WORKER = ROWS_PER_WORKER // CHUNK

_mesh = plsc.VectorSubcoreMesh(core_axis_name="c", subcore_axis_name="s")


NBUF = 10
OUTER = CHUNKS_PER_WORKER // NBUF


@functools.partial(
    pl.kernel,
    out_type=jax.ShapeDtypeStruct((TOTAL_ROWS, EMBED_DIM), jnp.float32),
    mesh=_mesh,
    scratch_types=[
        pltpu.VMEM((CHUNKS_PER_WORKER, CHUNK), jnp.int32),
        pltpu.VMEM((NBUF, CHUNK, EMBED_DIM), jnp.float32),
        pltpu.SemaphoreType.DMA,
        pltpu.SemaphoreType.DMA,
    ],
    compiler_params=pltpu.CompilerParams(use_tc_tiling_on_sc=False),
)
def _embed_sc(idx_hbm, table_hbm, out_hbm, idx_v, rows_v, gsem, wsem):
    wid = lax.axis_index("s") * NUM_CORES + lax.axis_index("c")
    base = wid * ROWS_PER_WORKER
    pltpu.sync_copy(idx_hbm.at[wid], idx_v)

    def wait_gather(b):
        pltpu.make_async_copy(
            table_hbm.at[idx_v.at[b]], rows_v.at[b], gsem).wait()

    def issue_write(b, j):
        pltpu.async_copy(rows_v.at[b],
                         out_hbm.at[pl.ds(base + j * CHUNK, CHUNK)], wsem)

    def wait_write(b, j):
        pltpu.make_async_copy(rows_v.at[b],
                              out_hbm.at[pl.ds(base + j * CHUNK, CHUNK)],
                              wsem).wait()

    for b in range(NBUF):
        pltpu.async_copy(table_hbm.at[idx_v.at[b]], rows_v.at[b], gsem)

    wait_gather(0)
    issue_write(0, 0)

    def body(i, carry):
        for b in range(NBUF):
            j = 1 + i * NBUF + b
            bj = (b + 1) % NBUF
            bp = b
            wait_gather(bj)
            issue_write(bj, j)
            wait_write(bp, j - 1)
            pltpu.async_copy(
                table_hbm.at[idx_v.at[j - 1 + NBUF]], rows_v.at[bp], gsem)
        return carry

    lax.fori_loop(0, (CHUNKS_PER_WORKER - NBUF) // NBUF, body, 0)

    for j in range(CHUNKS_PER_WORKER - NBUF + 1, CHUNKS_PER_WORKER):
        bj = j % NBUF
        wait_gather(bj)
        issue_write(bj, j)
        wait_write((j - 1) % NBUF, j - 1)
    wait_write((CHUNKS_PER_WORKER - 1) % NBUF, CHUNKS_PER_WORKER - 1)


def kernel(input_ids, table):
    flat = input_ids.reshape(-1).astype(jnp.int32)
    idx3d = flat.reshape(NUM_WORKERS, CHUNKS_PER_WORKER, CHUNK)
    out = _embed_sc(idx3d, table)
    embeds = out.reshape(input_ids.shape[0], input_ids.shape[1], EMBED_DIM)
    return (embeds, embeds, embeds)
